# Initial kernel scaffold; baseline (speedup 1.0000x reference)
import jax
import jax.numpy as jnp
from jax.experimental import pallas as pl


def kernel(x, edge_index, edge_type, weight_bases, w_comp, self_loop_weight, bias):
    raise NotImplementedError("write your pallas kernel here")


# speedup vs baseline: 2.1273x; 2.1273x over previous
import functools

import jax
import jax.numpy as jnp
from jax import lax
from jax.experimental import pallas as pl
from jax.experimental.pallas import tpu as pltpu
from jax.experimental.pallas import tpu_sc as plsc

N = 10000
E = 320000
D = 128
R = 32
NB = 8

NC = 2
NS = 16
NW = NC * NS

K = 128
BATCHES_PER_W = 79
E_PAD = NW * BATCHES_PER_W * K
NPAD = 10240
ROWS_PER_TILE = NPAD // NS


def _tc_xw_body(x_ref, wc_ref, bases_ref, out_ref):
    w = wc_ref[0, 0, 0] * bases_ref[0]
    for b in range(1, NB):
        w += wc_ref[0, 0, b] * bases_ref[b]
    out_ref[0] = jnp.dot(x_ref[...], w, preferred_element_type=jnp.float32)


def _tc_xw(x, w_comp, weight_bases):
    return pl.pallas_call(
        _tc_xw_body,
        grid=(R,),
        in_specs=[
            pl.BlockSpec((N, D), lambda r: (0, 0)),
            pl.BlockSpec((1, 1, NB), lambda r: (r, 0, 0)),
            pl.BlockSpec((NB, D, D), lambda r: (0, 0, 0)),
        ],
        out_specs=pl.BlockSpec((1, N, D), lambda r: (r, 0, 0)),
        out_shape=jax.ShapeDtypeStruct((R, N, D), jnp.float32),
    )(x, w_comp.reshape(R, 1, NB), weight_bases)


def _sc_agg_body(src_hbm, dst_hbm, rel_hbm, zeros_hbm, xw_hbm, out_hbm,
                 srcb, dstb, relb, idxb, rows, agg, sem):
    c = lax.axis_index("c")
    s = lax.axis_index("s")
    wid = s * NC + c
    pltpu.sync_copy(zeros_hbm.at[pl.ds(s * ROWS_PER_TILE, ROWS_PER_TILE)],
                    agg.at[pl.ds(s * ROWS_PER_TILE, ROWS_PER_TILE)])
    plsc.subcore_barrier()

    ebase = wid * (BATCHES_PER_W * K)

    @pl.loop(0, BATCHES_PER_W)
    def _(i):
        off = ebase + i * K
        pltpu.sync_copy(src_hbm.at[pl.ds(off, K)], srcb)
        pltpu.sync_copy(rel_hbm.at[pl.ds(off, K)], relb)
        pltpu.sync_copy(dst_hbm.at[pl.ds(off, K)], dstb)
        for j in range(K // 16):
            sl = pl.ds(j * 16, 16)
            idxb[sl] = relb[sl] * N + srcb[sl]
        pltpu.async_copy(xw_hbm.at[idxb], rows, sem).wait()
        pltpu.sync_copy(rows, agg.at[dstb], add=True)

    plsc.subcore_barrier()
    pltpu.sync_copy(agg.at[pl.ds(s * ROWS_PER_TILE, ROWS_PER_TILE)],
                    out_hbm.at[c, pl.ds(s * ROWS_PER_TILE, ROWS_PER_TILE)])


def _sc_agg(src, dst, rel, zeros, xw_flat):
    mesh = plsc.VectorSubcoreMesh(core_axis_name="c", subcore_axis_name="s")
    return pl.kernel(
        _sc_agg_body,
        out_type=jax.ShapeDtypeStruct((NC, NPAD, D), jnp.float32),
        mesh=mesh,
        scratch_types=[
            pltpu.VMEM((K,), jnp.int32),
            pltpu.VMEM((K,), jnp.int32),
            pltpu.VMEM((K,), jnp.int32),
            pltpu.VMEM((K,), jnp.int32),
            pltpu.VMEM((K, D), jnp.float32),
            pltpu.VMEM_SHARED((NPAD, D), jnp.float32),
            pltpu.SemaphoreType.DMA,
        ],
    )(src, dst, rel, zeros, xw_flat)


def _tc_out_body(x_ref, w_ref, b_ref, agg_ref, out_ref):
    acc = jnp.dot(x_ref[...], w_ref[...], preferred_element_type=jnp.float32)
    out_ref[...] = jnp.maximum(acc + agg_ref[0] + agg_ref[1] + b_ref[...], 0.0)


def _tc_out(x, self_loop_weight, bias, agg_pair):
    blk = 1000
    return pl.pallas_call(
        _tc_out_body,
        grid=(N // blk,),
        in_specs=[
            pl.BlockSpec((blk, D), lambda i: (i, 0)),
            pl.BlockSpec((D, D), lambda i: (0, 0)),
            pl.BlockSpec((1, D), lambda i: (0, 0)),
            pl.BlockSpec((NC, blk, D), lambda i: (0, i, 0)),
        ],
        out_specs=pl.BlockSpec((blk, D), lambda i: (i, 0)),
        out_shape=jax.ShapeDtypeStruct((N, D), jnp.float32),
    )(x, self_loop_weight, bias.reshape(1, D), agg_pair)


def kernel(x, edge_index, edge_type, weight_bases, w_comp, self_loop_weight, bias):
    src = edge_index[0].astype(jnp.int32)
    dst = edge_index[1].astype(jnp.int32)
    rel = edge_type.astype(jnp.int32)
    pad = E_PAD - E
    src_p = jnp.concatenate([src, jnp.zeros((pad,), jnp.int32)])
    dst_p = jnp.concatenate([dst, jnp.full((pad,), N, jnp.int32)])
    rel_p = jnp.concatenate([rel, jnp.zeros((pad,), jnp.int32)])

    xw = _tc_xw(x, w_comp, weight_bases)
    zeros = jnp.zeros((NPAD, D), jnp.float32)
    agg_pair = _sc_agg(src_p, dst_p, rel_p, zeros, xw.reshape(R * N, D))
    return _tc_out(x, self_loop_weight, bias, agg_pair)


# speedup vs baseline: 2.3268x; 1.0938x over previous
import functools

import jax
import jax.numpy as jnp
from jax import lax
from jax.experimental import pallas as pl
from jax.experimental.pallas import tpu as pltpu
from jax.experimental.pallas import tpu_sc as plsc

N = 10000
E = 320000
D = 128
R = 32
NB = 8

NC = 2
NS = 16
NW = NC * NS

K = 128
CB = 16
NCHUNK = 5
NBATCH = NCHUNK * CB
E_PAD = NW * NBATCH * K
NPAD = 10240
ROWS_PER_TILE = NPAD // NS


def _tc_xw_body(x_ref, wc_ref, bases_ref, out_ref):
    w = wc_ref[0, 0, 0] * bases_ref[0]
    for b in range(1, NB):
        w += wc_ref[0, 0, b] * bases_ref[b]
    out_ref[0] = jnp.dot(x_ref[...], w, preferred_element_type=jnp.float32)


def _tc_xw(x, w_comp, weight_bases):
    return pl.pallas_call(
        _tc_xw_body,
        grid=(R,),
        in_specs=[
            pl.BlockSpec((N, D), lambda r: (0, 0)),
            pl.BlockSpec((1, 1, NB), lambda r: (r, 0, 0)),
            pl.BlockSpec((NB, D, D), lambda r: (0, 0, 0)),
        ],
        out_specs=pl.BlockSpec((1, N, D), lambda r: (r, 0, 0)),
        out_shape=jax.ShapeDtypeStruct((R, N, D), jnp.float32),
    )(x, w_comp.reshape(R, 1, NB), weight_bases)


def _sc_agg_body(src_hbm, dst_hbm, rel_hbm, zeros_hbm, xw_hbm, out_hbm,
                 srcb, dstb, relb, idxb, rows0, rows1, agg, sem0, sem1):
    c = lax.axis_index("c")
    s = lax.axis_index("s")
    wid = s * NC + c
    pltpu.sync_copy(zeros_hbm.at[pl.ds(s * ROWS_PER_TILE, ROWS_PER_TILE)],
                    agg.at[pl.ds(s * ROWS_PER_TILE, ROWS_PER_TILE)])
    plsc.subcore_barrier()

    rows = (rows0, rows1)
    sems = (sem0, sem1)

    @pl.loop(0, NCHUNK)
    def _(cc):
        csl = pl.ds(cc * CB, CB)
        pltpu.sync_copy(src_hbm.at[wid, csl], srcb)
        pltpu.sync_copy(rel_hbm.at[wid, csl], relb)
        pltpu.sync_copy(dst_hbm.at[wid, csl], dstb)
        for i in range(CB):
            for j in range(K // 16):
                sl = pl.ds(j * 16, 16)
                idxb[i, sl] = relb[i, sl] * N + srcb[i, sl]
        pltpu.async_copy(xw_hbm.at[idxb.at[0]], rows[0], sems[0])
        pltpu.async_copy(xw_hbm.at[idxb.at[1]], rows[1], sems[1])
        for i in range(CB):
            p = i % 2
            pltpu.make_async_copy(xw_hbm.at[idxb.at[i]], rows[p], sems[p]).wait()
            pltpu.sync_copy(rows[p], agg.at[dstb.at[i]], add=True)
            if i + 2 < CB:
                pltpu.async_copy(xw_hbm.at[idxb.at[i + 2]], rows[p], sems[p])

    plsc.subcore_barrier()
    pltpu.sync_copy(agg.at[pl.ds(s * ROWS_PER_TILE, ROWS_PER_TILE)],
                    out_hbm.at[c, pl.ds(s * ROWS_PER_TILE, ROWS_PER_TILE)])


def _sc_agg(src, dst, rel, zeros, xw_flat):
    mesh = plsc.VectorSubcoreMesh(core_axis_name="c", subcore_axis_name="s")
    return pl.kernel(
        _sc_agg_body,
        out_type=jax.ShapeDtypeStruct((NC, NPAD, D), jnp.float32),
        mesh=mesh,
        scratch_types=[
            pltpu.VMEM((CB, K), jnp.int32),
            pltpu.VMEM((CB, K), jnp.int32),
            pltpu.VMEM((CB, K), jnp.int32),
            pltpu.VMEM((CB, K), jnp.int32),
            pltpu.VMEM((K, D), jnp.float32),
            pltpu.VMEM((K, D), jnp.float32),
            pltpu.VMEM_SHARED((NPAD, D), jnp.float32),
            pltpu.SemaphoreType.DMA,
            pltpu.SemaphoreType.DMA,
        ],
    )(src, dst, rel, zeros, xw_flat)


def _tc_out_body(x_ref, w_ref, b_ref, agg_ref, out_ref):
    acc = jnp.dot(x_ref[...], w_ref[...], preferred_element_type=jnp.float32)
    out_ref[...] = jnp.maximum(acc + agg_ref[0] + agg_ref[1] + b_ref[...], 0.0)


def _tc_out(x, self_loop_weight, bias, agg_pair):
    blk = 1000
    return pl.pallas_call(
        _tc_out_body,
        grid=(N // blk,),
        in_specs=[
            pl.BlockSpec((blk, D), lambda i: (i, 0)),
            pl.BlockSpec((D, D), lambda i: (0, 0)),
            pl.BlockSpec((1, D), lambda i: (0, 0)),
            pl.BlockSpec((NC, blk, D), lambda i: (0, i, 0)),
        ],
        out_specs=pl.BlockSpec((blk, D), lambda i: (i, 0)),
        out_shape=jax.ShapeDtypeStruct((N, D), jnp.float32),
    )(x, self_loop_weight, bias.reshape(1, D), agg_pair)


def kernel(x, edge_index, edge_type, weight_bases, w_comp, self_loop_weight, bias):
    src = edge_index[0].astype(jnp.int32)
    dst = edge_index[1].astype(jnp.int32)
    rel = edge_type.astype(jnp.int32)
    pad = E_PAD - E
    src_p = jnp.concatenate([src, jnp.zeros((pad,), jnp.int32)]).reshape(NW, NBATCH, K)
    dst_p = jnp.concatenate([dst, jnp.full((pad,), N, jnp.int32)]).reshape(NW, NBATCH, K)
    rel_p = jnp.concatenate([rel, jnp.zeros((pad,), jnp.int32)]).reshape(NW, NBATCH, K)

    xw = _tc_xw(x, w_comp, weight_bases)
    zeros = jnp.zeros((NPAD, D), jnp.float32)
    agg_pair = _sc_agg(src_p, dst_p, rel_p, zeros, xw.reshape(R * N, D))
    return _tc_out(x, self_loop_weight, bias, agg_pair)


# speedup vs baseline: 2.4209x; 1.0404x over previous
import functools

import jax
import jax.numpy as jnp
from jax import lax
from jax.experimental import pallas as pl
from jax.experimental.pallas import tpu as pltpu
from jax.experimental.pallas import tpu_sc as plsc

N = 10000
E = 320000
D = 128
R = 32
NB = 8

NC = 2
NS = 16

K = 128
CB = 8
B0 = 120
B1 = 40
NCHUNK0 = B0 // CB
NCHUNK1 = B1 // CB
BPAIR = B0 + B1
NROWS = NS * BPAIR
E_PAD = NROWS * K
NPAD = 10112
ROWS_PER_TILE = NPAD // NS


def _tc_xw_body(x_ref, wc_ref, bases_ref, out_ref):
    w = wc_ref[0, 0, 0] * bases_ref[0]
    for b in range(1, NB):
        w += wc_ref[0, 0, b] * bases_ref[b]
    out_ref[0] = jnp.dot(x_ref[...], w, preferred_element_type=jnp.float32)


def _tc_xw(x, w_comp, weight_bases):
    return pl.pallas_call(
        _tc_xw_body,
        grid=(R,),
        in_specs=[
            pl.BlockSpec((N, D), lambda r: (0, 0)),
            pl.BlockSpec((1, 1, NB), lambda r: (r, 0, 0)),
            pl.BlockSpec((NB, D, D), lambda r: (0, 0, 0)),
        ],
        out_specs=pl.BlockSpec((1, N, D), lambda r: (r, 0, 0)),
        out_shape=jax.ShapeDtypeStruct((R, N, D), jnp.float32),
    )(x, w_comp.reshape(R, 1, NB), weight_bases)


def _sc_agg_body(src_hbm, dst_hbm, rel_hbm, xw_hbm, out_hbm,
                 srcb, dstb, relb, idxb, rows0, rows1, agg, sem0, sem1):
    c = lax.axis_index("c")
    s = lax.axis_index("s")

    @pl.loop(0, K)
    def _(i):
        for j in range(D // 16):
            rows0[i, pl.ds(j * 16, 16)] = jnp.zeros((16,), jnp.float32)

    for k in range(5):
        cnt = K if k < 4 else ROWS_PER_TILE - 4 * K
        pltpu.sync_copy(rows0.at[pl.ds(0, cnt)],
                        agg.at[pl.ds(s * ROWS_PER_TILE + k * K, cnt)])
    plsc.subcore_barrier()

    rows = (rows0, rows1)
    sems = (sem0, sem1)
    rowbase = s * BPAIR + c * B0
    nchunk = NCHUNK0 - (NCHUNK0 - NCHUNK1) * c

    @pl.loop(0, nchunk)
    def _(cc):
        csl = pl.ds(rowbase + cc * CB, CB)
        pltpu.sync_copy(src_hbm.at[csl], srcb)
        pltpu.sync_copy(rel_hbm.at[csl], relb)
        pltpu.sync_copy(dst_hbm.at[csl], dstb)
        for i in range(CB):
            for j in range(K // 16):
                sl = pl.ds(j * 16, 16)
                idxb[i, sl] = relb[i, sl] * N + srcb[i, sl]
        pltpu.async_copy(xw_hbm.at[idxb.at[0]], rows[0], sems[0])
        pltpu.async_copy(xw_hbm.at[idxb.at[1]], rows[1], sems[1])
        for i in range(CB):
            p = i % 2
            pltpu.make_async_copy(xw_hbm.at[idxb.at[i]], rows[p], sems[p]).wait()
            pltpu.sync_copy(rows[p], agg.at[dstb.at[i]], add=True)
            if i + 2 < CB:
                pltpu.async_copy(xw_hbm.at[idxb.at[i + 2]], rows[p], sems[p])

    plsc.subcore_barrier()
    pltpu.sync_copy(agg.at[pl.ds(s * ROWS_PER_TILE, ROWS_PER_TILE)],
                    out_hbm.at[c, pl.ds(s * ROWS_PER_TILE, ROWS_PER_TILE)])


def _sc_agg(src, dst, rel, xw_flat):
    mesh = plsc.VectorSubcoreMesh(core_axis_name="c", subcore_axis_name="s")
    return pl.kernel(
        _sc_agg_body,
        out_type=jax.ShapeDtypeStruct((NC, NPAD, D), jnp.float32),
        mesh=mesh,
        scratch_types=[
            pltpu.VMEM((CB, K), jnp.int32),
            pltpu.VMEM((CB, K), jnp.int32),
            pltpu.VMEM((CB, K), jnp.int32),
            pltpu.VMEM((CB, K), jnp.int32),
            pltpu.VMEM((K, D), jnp.float32),
            pltpu.VMEM((K, D), jnp.float32),
            pltpu.VMEM_SHARED((NPAD, D), jnp.float32),
            pltpu.SemaphoreType.DMA,
            pltpu.SemaphoreType.DMA,
        ],
    )(src, dst, rel, xw_flat)


def _tc_out_body(x_ref, w_ref, b_ref, agg_ref, out_ref):
    acc = jnp.dot(x_ref[...], w_ref[...], preferred_element_type=jnp.float32)
    out_ref[...] = jnp.maximum(acc + agg_ref[0] + agg_ref[1] + b_ref[...], 0.0)


def _tc_out(x, self_loop_weight, bias, agg_pair):
    blk = 1000
    return pl.pallas_call(
        _tc_out_body,
        grid=(N // blk,),
        in_specs=[
            pl.BlockSpec((blk, D), lambda i: (i, 0)),
            pl.BlockSpec((D, D), lambda i: (0, 0)),
            pl.BlockSpec((1, D), lambda i: (0, 0)),
            pl.BlockSpec((NC, blk, D), lambda i: (0, i, 0)),
        ],
        out_specs=pl.BlockSpec((blk, D), lambda i: (i, 0)),
        out_shape=jax.ShapeDtypeStruct((N, D), jnp.float32),
    )(x, self_loop_weight, bias.reshape(1, D), agg_pair)


def kernel(x, edge_index, edge_type, weight_bases, w_comp, self_loop_weight, bias):
    src = edge_index[0].astype(jnp.int32)
    dst = edge_index[1].astype(jnp.int32)
    rel = edge_type.astype(jnp.int32)
    pad = E_PAD - E
    src_p = jnp.concatenate([src, jnp.zeros((pad,), jnp.int32)]).reshape(NROWS, K)
    dst_p = jnp.concatenate([dst, jnp.full((pad,), N, jnp.int32)]).reshape(NROWS, K)
    rel_p = jnp.concatenate([rel, jnp.zeros((pad,), jnp.int32)]).reshape(NROWS, K)

    xw = _tc_xw(x, w_comp, weight_bases)
    agg_pair = _sc_agg(src_p, dst_p, rel_p, xw.reshape(R * N, D))
    return _tc_out(x, self_loop_weight, bias, agg_pair)
